# baseline (speedup 1.0000x reference)
import functools

import jax
import jax.numpy as jnp
from jax import lax
from jax.experimental import pallas as pl
from jax.experimental.pallas import tpu as pltpu
from jax.experimental.pallas import tpu_sc as plsc

M = 100000
F = 64
B = 1024
C = 1000
CP = 1024
TEMP = 0.05

NC = 2
NS = 16
NW = NC * NS
CHUNK = 128
NCHUNK = 25
NBUF = 4
RW = CHUNK * NCHUNK
MP = RW * NW
PART_BASE = (M // CHUNK) * CHUNK
PART_LEN = M - PART_BASE

IDX_PER_W = B // NW
STRIPE = CP // NS


def _sc_body(feat_hbm, labels_hbm, lab3_hbm, idx_hbm,
             sums_hbm, cnts_hbm, tgt_hbm,
             acc_sh, cnt_sh,
             rows0_v, rows1_v, rows2_v, rows3_v, lab_v, ones_v,
             zrow_v, zcnt_v, idx_v, tgt_v,
             fsem0, fsem1, fsem2, fsem3, csem):
    cid = lax.axis_index("c")
    sid = lax.axis_index("s")
    wid = sid * NC + cid
    bufs = (rows0_v, rows1_v, rows2_v, rows3_v)
    sems = (fsem0, fsem1, fsem2, fsem3)

    def fill_z(i, _):
        for j in range(F // 16):
            zrow_v[i, pl.ds(j * 16, 16)] = jnp.zeros((16,), jnp.float32)
        zcnt_v[i] = jnp.zeros((16,), jnp.float32)
        return 0
    lax.fori_loop(0, STRIPE, fill_z, 0)

    def fill_o(i, _):
        ones_v[i] = jnp.ones((16,), jnp.float32)
        return 0
    lax.fori_loop(0, CHUNK, fill_o, 0)

    pltpu.sync_copy(zrow_v, acc_sh.at[pl.ds(sid * STRIPE, STRIPE)])
    pltpu.sync_copy(zcnt_v, cnt_sh.at[pl.ds(sid * STRIPE, STRIPE)])

    pltpu.sync_copy(lab3_hbm.at[wid], lab_v)

    plsc.subcore_barrier()

    def fetch_descr(t):
        base = wid * RW + t * CHUNK
        buf, sem = bufs[t % NBUF], sems[t % NBUF]
        full = pltpu.make_async_copy(feat_hbm.at[pl.ds(base, CHUNK)], buf, sem)
        part = pltpu.make_async_copy(feat_hbm.at[pl.ds(base, PART_LEN)],
                                     buf.at[pl.ds(0, PART_LEN)], sem)
        return base, full, part

    def start_fetch(t):
        base, full, part = fetch_descr(t)
        pl.when(base + CHUNK <= M)(full.start)
        pl.when(base == PART_BASE)(part.start)

    def wait_fetch(t):
        base, full, part = fetch_descr(t)
        pl.when(base + CHUNK <= M)(full.wait)
        pl.when(base == PART_BASE)(part.wait)

    cdescs = []
    for t in range(NCHUNK):
        d = pltpu.async_copy(ones_v, cnt_sh.at[lab_v.at[t]], csem, add=True)
        cdescs.append(d)

    for t in range(min(NBUF, NCHUNK)):
        start_fetch(t)
    for t in range(NCHUNK):
        wait_fetch(t)
        pltpu.sync_copy(bufs[t % NBUF], acc_sh.at[lab_v.at[t]], add=True)
        if t + NBUF < NCHUNK:
            start_fetch(t + NBUF)

    for d in cdescs:
        d.wait()

    pltpu.sync_copy(idx_hbm.at[pl.ds(wid * IDX_PER_W, IDX_PER_W)], idx_v)
    pltpu.sync_copy(labels_hbm.at[idx_v], tgt_v)
    pltpu.sync_copy(tgt_v, tgt_hbm.at[pl.ds(wid * IDX_PER_W, IDX_PER_W)])

    plsc.subcore_barrier()

    pltpu.sync_copy(acc_sh.at[pl.ds(sid * STRIPE, STRIPE)],
                    sums_hbm.at[cid, pl.ds(sid * STRIPE, STRIPE)])
    pltpu.sync_copy(cnt_sh.at[pl.ds(sid * STRIPE, STRIPE)],
                    cnts_hbm.at[cid, pl.ds(sid * STRIPE, STRIPE)])


@functools.cache
def _sc_segment():
    return pl.kernel(
        _sc_body,
        out_type=(
            jax.ShapeDtypeStruct((NC, CP, F), jnp.float32),
            jax.ShapeDtypeStruct((NC, CP, 16), jnp.float32),
            jax.ShapeDtypeStruct((B,), jnp.int32),
        ),
        mesh=plsc.VectorSubcoreMesh(
            core_axis_name="c", subcore_axis_name="s",
            num_cores=NC, num_subcores=NS),
        scratch_types=(
            pltpu.VMEM_SHARED((CP, F), jnp.float32),
            pltpu.VMEM_SHARED((CP, 16), jnp.float32),
            pltpu.VMEM((CHUNK, F), jnp.float32),
            pltpu.VMEM((CHUNK, F), jnp.float32),
            pltpu.VMEM((CHUNK, F), jnp.float32),
            pltpu.VMEM((CHUNK, F), jnp.float32),
            pltpu.VMEM((NCHUNK, CHUNK), jnp.int32),
            pltpu.VMEM((CHUNK, 16), jnp.float32),
            pltpu.VMEM((STRIPE, F), jnp.float32),
            pltpu.VMEM((STRIPE, 16), jnp.float32),
            pltpu.VMEM((IDX_PER_W,), jnp.int32),
            pltpu.VMEM((IDX_PER_W,), jnp.int32),
            pltpu.SemaphoreType.DMA,
            pltpu.SemaphoreType.DMA,
            pltpu.SemaphoreType.DMA,
            pltpu.SemaphoreType.DMA,
            pltpu.SemaphoreType.DMA,
        ),
    )


TXBLK = 512


def _tx_body(ft_ref, out_ref):
    a = ft_ref[...]
    ri = lax.broadcasted_iota(jnp.int32, (F, F), 0)
    ci = lax.broadcasted_iota(jnp.int32, (F, F), 1)
    eye = (ri == ci).astype(jnp.float32)
    out_ref[...] = lax.dot_general(a, eye, (((0,), (0,)), ((), ())),
                                   precision=lax.Precision.HIGHEST,
                                   preferred_element_type=jnp.float32)


def _transpose_features(features):
    feat_t = jnp.swapaxes(features, 0, 1)
    grid = (M + TXBLK - 1) // TXBLK
    return pl.pallas_call(
        _tx_body,
        grid=(grid,),
        in_specs=[pl.BlockSpec((F, TXBLK), lambda i: (0, i))],
        out_specs=pl.BlockSpec((TXBLK, F), lambda i: (i, 0)),
        out_shape=jax.ShapeDtypeStruct((M, F), jnp.float32),
    )(feat_t)


def _tc_body(x_ref, sums_ref, cnts_ref, tgt_ref, out_ref):
    x = x_ref[...]
    s2 = jnp.sum(x * x, axis=1, keepdims=True)
    xn = x / jnp.maximum(jnp.sqrt(s2), 1e-12)

    S = sums_ref[0] + sums_ref[1]
    n = cnts_ref[0][:, 0:1] + cnts_ref[1][:, 0:1]

    ci = lax.broadcasted_iota(jnp.int32, (CP, 1), 0)
    valid = (ci < C) & (n > 0.0)
    scale = jnp.where(valid, 1.0 / (TEMP * n), 0.0)
    Ms = jnp.where(valid, S, 0.0) * scale
    bias = jnp.where(valid, 0.0, -1e30)

    onec = jnp.ones((B, 1), jnp.float32)
    zpadx = jnp.zeros((B, 128 - F - 1), jnp.float32)
    zpadm = jnp.zeros((CP, 128 - F - 1), jnp.float32)
    xa = jnp.concatenate([xn, onec, zpadx], axis=1)
    ma = jnp.concatenate([Ms, bias, zpadm], axis=1)

    vec = lax.dot_general(xa, ma, (((1,), (1,)), ((), ())),
                          precision=lax.Precision.HIGHEST,
                          preferred_element_type=jnp.float32)

    exps = jnp.exp(vec)
    denom = jnp.sum(exps, axis=1, keepdims=True) + 1e-6

    tj = lax.broadcasted_iota(jnp.int32, (B, CP), 1)
    th = tgt_ref[...]
    pt = jnp.sum(jnp.where(tj == th, exps, 0.0), axis=1, keepdims=True)
    lt = jnp.log(pt / denom + 1e-6)
    out_ref[...] = -jnp.sum(lt, axis=0, keepdims=True) / B


def kernel(inputs, indexes, features, labels):
    labels = labels.astype(jnp.int32)
    indexes = indexes.astype(jnp.int32)
    lab_pad = jnp.concatenate(
        [labels, jnp.full((MP - M,), C, jnp.int32)]).reshape(NW, NCHUNK, CHUNK)

    feat_rows = _transpose_features(features)
    sums, cnts, targets = _sc_segment()(feat_rows, labels, lab_pad, indexes)

    loss = pl.pallas_call(
        _tc_body,
        out_shape=jax.ShapeDtypeStruct((1, 1), jnp.float32),
    )(inputs, sums, cnts, targets.reshape(B, 1))
    return loss.reshape(())


# speedup vs baseline: 1.8750x; 1.8750x over previous
import functools

import jax
import jax.numpy as jnp
from jax import lax
from jax.experimental import pallas as pl
from jax.experimental.pallas import tpu as pltpu
from jax.experimental.pallas import tpu_sc as plsc

M = 100000
F = 64
B = 1024
C = 1000
CP = 1024
TEMP = 0.05

NC = 2
NS = 16
NW = NC * NS
CHUNK = 128
NBUF = 4
NCHUNK = 25
MP = CHUNK * NCHUNK * NW
CQ0 = 13
CQ1 = NCHUNK - CQ0
SPLIT = CQ0 * CHUNK * NW

IDX_PER_W = B // NW
STRIPE = CP // NS


def _make_sc_body(cq, valid_rows, do_targets):
    rw = cq * CHUNK
    full_slice = valid_rows >= rw * NW
    part_base = (valid_rows // CHUNK) * CHUNK
    part_len = valid_rows - part_base

    def body(*refs):
        if do_targets:
            (feat_hbm, labels_hbm, lab3_hbm, idx_hbm,
             sums_hbm, cnts_hbm, tgt_hbm,
             acc_sh, cnt_sh,
             rows0_v, rows1_v, rows2_v, rows3_v, lab_v, ones_v,
             zrow_v, zcnt_v, idx_v, tgt_v,
             fsem0, fsem1, fsem2, fsem3, csem) = refs
        else:
            (feat_hbm, lab3_hbm,
             sums_hbm, cnts_hbm,
             acc_sh, cnt_sh,
             rows0_v, rows1_v, rows2_v, rows3_v, lab_v, ones_v,
             zrow_v, zcnt_v,
             fsem0, fsem1, fsem2, fsem3, csem) = refs
        cid = lax.axis_index("c")
        sid = lax.axis_index("s")
        wid = sid * NC + cid
        bufs = (rows0_v, rows1_v, rows2_v, rows3_v)
        sems = (fsem0, fsem1, fsem2, fsem3)

        def fill_z(i, _):
            for j in range(F // 16):
                zrow_v[i, pl.ds(j * 16, 16)] = jnp.zeros((16,), jnp.float32)
            zcnt_v[i] = jnp.zeros((16,), jnp.float32)
            return 0
        lax.fori_loop(0, STRIPE, fill_z, 0)

        def fill_o(i, _):
            ones_v[i] = jnp.ones((16,), jnp.float32)
            return 0
        lax.fori_loop(0, CHUNK, fill_o, 0)

        pltpu.sync_copy(zrow_v, acc_sh.at[pl.ds(sid * STRIPE, STRIPE)])
        pltpu.sync_copy(zcnt_v, cnt_sh.at[pl.ds(sid * STRIPE, STRIPE)])

        pltpu.sync_copy(lab3_hbm.at[wid], lab_v)

        plsc.subcore_barrier()

        def fetch_descr(t):
            base = wid * rw + t * CHUNK
            buf, sem = bufs[t % NBUF], sems[t % NBUF]
            full = pltpu.make_async_copy(
                feat_hbm.at[pl.ds(base, CHUNK)], buf, sem)
            if part_len:
                part = pltpu.make_async_copy(
                    feat_hbm.at[pl.ds(base, part_len)],
                    buf.at[pl.ds(0, part_len)], sem)
            else:
                part = None
            return base, full, part

        def start_fetch(t):
            base, full, part = fetch_descr(t)
            if full_slice:
                full.start()
            else:
                pl.when(base + CHUNK <= valid_rows)(full.start)
                if part is not None:
                    pl.when(base == part_base)(part.start)

        def wait_fetch(t):
            base, full, part = fetch_descr(t)
            if full_slice:
                full.wait()
            else:
                pl.when(base + CHUNK <= valid_rows)(full.wait)
                if part is not None:
                    pl.when(base == part_base)(part.wait)

        cdescs = []
        for t in range(cq):
            d = pltpu.async_copy(ones_v, cnt_sh.at[lab_v.at[t]], csem,
                                 add=True)
            cdescs.append(d)

        for t in range(min(NBUF, cq)):
            start_fetch(t)
        for t in range(cq):
            wait_fetch(t)
            pltpu.sync_copy(bufs[t % NBUF], acc_sh.at[lab_v.at[t]], add=True)
            if t + NBUF < cq:
                start_fetch(t + NBUF)

        for d in cdescs:
            d.wait()

        if do_targets:
            pltpu.sync_copy(idx_hbm.at[pl.ds(wid * IDX_PER_W, IDX_PER_W)],
                            idx_v)
            pltpu.sync_copy(labels_hbm.at[idx_v], tgt_v)
            pltpu.sync_copy(tgt_v, tgt_hbm.at[pl.ds(wid * IDX_PER_W,
                                                    IDX_PER_W)])

        plsc.subcore_barrier()

        pltpu.sync_copy(acc_sh.at[pl.ds(sid * STRIPE, STRIPE)],
                        sums_hbm.at[cid, pl.ds(sid * STRIPE, STRIPE)])
        pltpu.sync_copy(cnt_sh.at[pl.ds(sid * STRIPE, STRIPE)],
                        cnts_hbm.at[cid, pl.ds(sid * STRIPE, STRIPE)])

    return body


def _sc_outs(do_targets):
    outs = [jax.ShapeDtypeStruct((NC, CP, F), jnp.float32),
            jax.ShapeDtypeStruct((NC, CP, 16), jnp.float32)]
    if do_targets:
        outs.append(jax.ShapeDtypeStruct((B,), jnp.int32))
    return tuple(outs)


def _sc_scratch(cq):
    return (
        pltpu.VMEM_SHARED((CP, F), jnp.float32),
        pltpu.VMEM_SHARED((CP, 16), jnp.float32),
        pltpu.VMEM((CHUNK, F), jnp.float32),
        pltpu.VMEM((CHUNK, F), jnp.float32),
        pltpu.VMEM((CHUNK, F), jnp.float32),
        pltpu.VMEM((CHUNK, F), jnp.float32),
        pltpu.VMEM((cq, CHUNK), jnp.int32),
        pltpu.VMEM((CHUNK, 16), jnp.float32),
        pltpu.VMEM((STRIPE, F), jnp.float32),
        pltpu.VMEM((STRIPE, 16), jnp.float32),
    )


def _sc_tgt_scratch():
    return (
        pltpu.VMEM((IDX_PER_W,), jnp.int32),
        pltpu.VMEM((IDX_PER_W,), jnp.int32),
    )


def _sems():
    return (pltpu.SemaphoreType.DMA,) * 5


@functools.cache
def _sc_slice0():
    valid = min(M, SPLIT)
    return pl.kernel(
        _make_sc_body(CQ0, valid, True),
        out_type=_sc_outs(True),
        mesh=plsc.VectorSubcoreMesh(
            core_axis_name="c", subcore_axis_name="s",
            num_cores=NC, num_subcores=NS),
        scratch_types=_sc_scratch(CQ0) + _sc_tgt_scratch() + _sems(),
    )


@functools.cache
def _sc_slice1():
    valid = M - SPLIT
    return pl.kernel(
        _make_sc_body(CQ1, valid, False),
        out_type=_sc_outs(False),
        mesh=plsc.VectorSubcoreMesh(
            core_axis_name="c", subcore_axis_name="s",
            num_cores=NC, num_subcores=NS),
        scratch_types=_sc_scratch(CQ1) + _sems(),
    )


def _tc_body(x_ref, s0_ref, s1_ref, c0_ref, c1_ref, tgt_ref, out_ref):
    x = x_ref[...]
    s2 = jnp.sum(x * x, axis=1, keepdims=True)
    xn = x / jnp.maximum(jnp.sqrt(s2), 1e-12)

    S = s0_ref[0] + s0_ref[1] + s1_ref[0] + s1_ref[1]
    nw = c0_ref[0] + c0_ref[1] + c1_ref[0] + c1_ref[1]
    n = nw[:, 0:1]

    ci = lax.broadcasted_iota(jnp.int32, (CP, 1), 0)
    valid = (ci < C) & (n > 0.0)
    scale = jnp.where(valid, 1.0 / (TEMP * n), 0.0)
    Ms = jnp.where(valid, S, 0.0) * scale
    bias = jnp.where(valid, 0.0, -1e30)

    onec = jnp.ones((B, 1), jnp.float32)
    zpadx = jnp.zeros((B, 128 - F - 1), jnp.float32)
    zpadm = jnp.zeros((CP, 128 - F - 1), jnp.float32)
    xa = jnp.concatenate([xn, onec, zpadx], axis=1)
    ma = jnp.concatenate([Ms, bias, zpadm], axis=1)

    vec = lax.dot_general(xa, ma, (((1,), (1,)), ((), ())),
                          precision=lax.Precision.HIGHEST,
                          preferred_element_type=jnp.float32)

    exps = jnp.exp(vec)
    denom = jnp.sum(exps, axis=1, keepdims=True) + 1e-6

    tj = lax.broadcasted_iota(jnp.int32, (B, CP), 1)
    th = tgt_ref[...]
    pt = jnp.sum(jnp.where(tj == th, exps, 0.0), axis=1, keepdims=True)
    lt = jnp.log(pt / denom + 1e-6)
    out_ref[...] = -jnp.sum(lt, axis=0, keepdims=True) / B


def kernel(inputs, indexes, features, labels):
    labels = labels.astype(jnp.int32)
    indexes = indexes.astype(jnp.int32)
    lab_pad = jnp.concatenate(
        [labels, jnp.full((MP - M,), C, jnp.int32)])
    lab0 = lab_pad[:SPLIT].reshape(NW, CQ0, CHUNK)
    lab1 = lab_pad[SPLIT:].reshape(NW, CQ1, CHUNK)
    feat0 = lax.slice_in_dim(features, 0, SPLIT)
    feat1 = lax.slice_in_dim(features, SPLIT, M)

    sums0, cnts0, targets = _sc_slice0()(feat0, labels, lab0, indexes)
    sums1, cnts1 = _sc_slice1()(feat1, lab1)

    loss = pl.pallas_call(
        _tc_body,
        out_shape=jax.ShapeDtypeStruct((1, 1), jnp.float32),
    )(inputs, sums0, sums1, cnts0, cnts1, targets.reshape(B, 1))
    return loss.reshape(())


# speedup vs baseline: 2.2091x; 1.1782x over previous
import functools

import jax
import jax.numpy as jnp
from jax import lax
from jax.experimental import pallas as pl
from jax.experimental.pallas import tpu as pltpu
from jax.experimental.pallas import tpu_sc as plsc

M = 100000
F = 64
B = 1024
C = 1000
CP = 1024
TEMP = 0.05

NC = 2
NS = 16
NW = NC * NS
CHUNK = 128
NCHUNK = 25
NBUF = 4
RW = CHUNK * NCHUNK
MP = RW * NW
PART_BASE = (M // CHUNK) * CHUNK
PART_LEN = M - PART_BASE

IDX_PER_W = B // NW
STRIPE = CP // NS


def _sc_body(feat_hbm, labels_hbm, lab3_hbm, idx_hbm,
             sums_hbm, cnts_hbm, tgt_hbm,
             acc_sh, cnt_sh,
             rows0_v, rows1_v, rows2_v, rows3_v, lab_v, ones_v,
             zrow_v, zcnt_v, idx_v, tgt_v,
             fsem0, fsem1, fsem2, fsem3, csem, gsem, psem):
    cid = lax.axis_index("c")
    sid = lax.axis_index("s")
    wid = sid * NC + cid
    bufs = (rows0_v, rows1_v, rows2_v, rows3_v)
    sems = (fsem0, fsem1, fsem2, fsem3)

    d_idx = pltpu.async_copy(idx_hbm.at[pl.ds(wid * IDX_PER_W, IDX_PER_W)],
                             idx_v, gsem)

    def fill_z(i, _):
        for j in range(F // 16):
            zrow_v[i, pl.ds(j * 16, 16)] = jnp.zeros((16,), jnp.float32)
        zcnt_v[i] = jnp.zeros((16,), jnp.float32)
        return 0
    lax.fori_loop(0, STRIPE, fill_z, 0)

    def fill_o(i, _):
        ones_v[i] = jnp.ones((16,), jnp.float32)
        return 0
    lax.fori_loop(0, CHUNK, fill_o, 0)

    pltpu.sync_copy(zrow_v, acc_sh.at[pl.ds(sid * STRIPE, STRIPE)])
    pltpu.sync_copy(zcnt_v, cnt_sh.at[pl.ds(sid * STRIPE, STRIPE)])

    pltpu.sync_copy(lab3_hbm.at[wid], lab_v)

    d_idx.wait()

    plsc.subcore_barrier()

    d_tgt = pltpu.async_copy(labels_hbm.at[idx_v], tgt_v, gsem)

    def fetch_descr(t):
        base = wid * RW + t * CHUNK
        buf, sem = bufs[t % NBUF], sems[t % NBUF]
        full = pltpu.make_async_copy(feat_hbm.at[pl.ds(base, CHUNK)], buf, sem)
        part = pltpu.make_async_copy(feat_hbm.at[pl.ds(base, PART_LEN)],
                                     buf.at[pl.ds(0, PART_LEN)], sem)
        return base, full, part

    def start_fetch(t):
        base, full, part = fetch_descr(t)
        pl.when(base + CHUNK <= M)(full.start)
        pl.when(base == PART_BASE)(part.start)

    def wait_fetch(t):
        base, full, part = fetch_descr(t)
        pl.when(base + CHUNK <= M)(full.wait)
        pl.when(base == PART_BASE)(part.wait)

    cdescs = []
    for t in range(NCHUNK):
        cdescs.append(
            pltpu.async_copy(ones_v, cnt_sh.at[lab_v.at[t]], csem, add=True))

    for t in range(min(NBUF, NCHUNK)):
        start_fetch(t)
    for t in range(NCHUNK):
        wait_fetch(t)
        pltpu.sync_copy(bufs[t % NBUF], acc_sh.at[lab_v.at[t]], add=True)
        if t + NBUF < NCHUNK:
            start_fetch(t + NBUF)

    d_tgt.wait()
    d_wtgt = pltpu.async_copy(tgt_v, tgt_hbm.at[pl.ds(wid * IDX_PER_W,
                                                      IDX_PER_W)], gsem)
    for d in cdescs:
        d.wait()
    d_wtgt.wait()

    plsc.subcore_barrier()

    d_ps = pltpu.async_copy(acc_sh.at[pl.ds(sid * STRIPE, STRIPE)],
                            sums_hbm.at[cid, pl.ds(sid * STRIPE, STRIPE)],
                            psem)
    d_pc = pltpu.async_copy(cnt_sh.at[pl.ds(sid * STRIPE, STRIPE)],
                            cnts_hbm.at[cid, pl.ds(sid * STRIPE, STRIPE)],
                            psem)
    d_ps.wait()
    d_pc.wait()


@functools.cache
def _sc_segment():
    return pl.kernel(
        _sc_body,
        out_type=(
            jax.ShapeDtypeStruct((NC, CP, F), jnp.float32),
            jax.ShapeDtypeStruct((NC, CP, 16), jnp.float32),
            jax.ShapeDtypeStruct((B,), jnp.int32),
        ),
        mesh=plsc.VectorSubcoreMesh(
            core_axis_name="c", subcore_axis_name="s",
            num_cores=NC, num_subcores=NS),
        scratch_types=(
            pltpu.VMEM_SHARED((CP, F), jnp.float32),
            pltpu.VMEM_SHARED((CP, 16), jnp.float32),
            pltpu.VMEM((CHUNK, F), jnp.float32),
            pltpu.VMEM((CHUNK, F), jnp.float32),
            pltpu.VMEM((CHUNK, F), jnp.float32),
            pltpu.VMEM((CHUNK, F), jnp.float32),
            pltpu.VMEM((NCHUNK, CHUNK), jnp.int32),
            pltpu.VMEM((CHUNK, 16), jnp.float32),
            pltpu.VMEM((STRIPE, F), jnp.float32),
            pltpu.VMEM((STRIPE, 16), jnp.float32),
            pltpu.VMEM((IDX_PER_W,), jnp.int32),
            pltpu.VMEM((IDX_PER_W,), jnp.int32),
            pltpu.SemaphoreType.DMA,
            pltpu.SemaphoreType.DMA,
            pltpu.SemaphoreType.DMA,
            pltpu.SemaphoreType.DMA,
            pltpu.SemaphoreType.DMA,
            pltpu.SemaphoreType.DMA,
            pltpu.SemaphoreType.DMA,
        ),
    )


def _tc_body(x_ref, sums_ref, cnts_ref, tgt_ref, out_ref):
    x = x_ref[...]
    s2 = jnp.sum(x * x, axis=1, keepdims=True)
    xn = x / jnp.maximum(jnp.sqrt(s2), 1e-12)

    S = sums_ref[0] + sums_ref[1]
    n = cnts_ref[0][:, 0:1] + cnts_ref[1][:, 0:1]

    ci = lax.broadcasted_iota(jnp.int32, (CP, 1), 0)
    valid = (ci < C) & (n > 0.0)
    scale = jnp.where(valid, 1.0 / (TEMP * n), 0.0)
    Ms = jnp.where(valid, S, 0.0) * scale
    bias = jnp.where(valid, 0.0, -1e30)

    onec = jnp.ones((B, 1), jnp.float32)
    zpadx = jnp.zeros((B, 128 - F - 1), jnp.float32)
    zpadm = jnp.zeros((CP, 128 - F - 1), jnp.float32)
    xa = jnp.concatenate([xn, onec, zpadx], axis=1)
    ma = jnp.concatenate([Ms, bias, zpadm], axis=1)

    vec = lax.dot_general(xa, ma, (((1,), (1,)), ((), ())),
                          precision=lax.Precision.HIGHEST,
                          preferred_element_type=jnp.float32)

    exps = jnp.exp(vec)
    denom = jnp.sum(exps, axis=1, keepdims=True) + 1e-6

    tj = lax.broadcasted_iota(jnp.int32, (B, CP), 1)
    th = tgt_ref[...]
    pt = jnp.sum(jnp.where(tj == th, exps, 0.0), axis=1, keepdims=True)
    lt = jnp.log(pt / denom + 1e-6)
    out_ref[...] = -jnp.sum(lt, axis=0, keepdims=True) / B


def kernel(inputs, indexes, features, labels):
    labels = labels.astype(jnp.int32)
    indexes = indexes.astype(jnp.int32)
    lab_pad = jnp.concatenate(
        [labels, jnp.full((MP - M,), C, jnp.int32)]).reshape(NW, NCHUNK, CHUNK)

    sums, cnts, targets = _sc_segment()(features, labels, lab_pad, indexes)

    loss = pl.pallas_call(
        _tc_body,
        out_shape=jax.ShapeDtypeStruct((1, 1), jnp.float32),
    )(inputs, sums, cnts, targets.reshape(B, 1))
    return loss.reshape(())
